# baseline (speedup 1.0000x reference)
import functools
import jax
import jax.numpy as jnp
from jax import lax
from jax.experimental import pallas as pl
from jax.experimental.pallas import tpu as pltpu

H = 4
KNN = 32
EPS = 1e-5
BIG = 3.0e38
NEG = -3.0e38


def _ln(x, g, b):
    m = jnp.mean(x, axis=-1, keepdims=True)
    v = jnp.mean((x - m) * (x - m), axis=-1, keepdims=True)
    return (x - m) / jnp.sqrt(v + EPS) * g + b


def _kv_proj_kernel(pf_ref, w_ref, b_ref, g_ref, bb_ref, kp_ref, vp_ref, *, d):
    x = pf_ref[0]
    xn = _ln(x, g_ref[0][None, :], bb_ref[0][None, :])
    wk = w_ref[d:2 * d]
    wv = w_ref[2 * d:3 * d]
    bk = b_ref[0, d:2 * d][None, :]
    bv = b_ref[0, 2 * d:3 * d][None, :]
    kp_ref[0] = (lax.dot_general(xn, wk, (((1,), (1,)), ((), ())),
                                 preferred_element_type=jnp.float32)
                 + bk).astype(jnp.bfloat16)
    vp_ref[0] = (lax.dot_general(xn, wv, (((1,), (1,)), ((), ())),
                                 preferred_element_type=jnp.float32)
                 + bv).astype(jnp.bfloat16)


def _attn_kernel(q_ref, qc_ref, pc_ref, kp_ref, vp_ref,
                 w_ref, b_ref, ow_ref, ob_ref,
                 g1_ref, b1_ref, g2_ref, b2_ref,
                 fw1_ref, fb1_ref, fw2_ref, fb2_ref,
                 out_ref, *, d, ksel):
    dh = d // H
    scale = 1.0 / jnp.sqrt(jnp.float32(dh))

    qc = qc_ref[0]
    pc = pc_ref[0]
    pn = jnp.sum(pc * pc, axis=0, keepdims=True)
    cross = lax.dot_general(qc, pc, (((0,), (0,)), ((), ())),
                            preferred_element_type=jnp.float32)
    dist = pn - 2.0 * cross

    ncls = dist.shape[1]
    mcls = dist
    while ncls > 128:
        ncls //= 2
        mcls = jnp.minimum(mcls[:, :ncls], mcls[:, ncls:])
    lo = jnp.min(mcls, axis=1, keepdims=True)
    kf = jnp.float32(ksel)

    def ubody(_, carry):
        ulo, uhi = carry
        umid = 0.5 * (ulo + uhi)
        ucnt = jnp.sum(jnp.where(mcls <= umid, 1.0, 0.0), axis=1,
                       keepdims=True)
        uge = ucnt >= kf
        return jnp.where(uge, ulo, umid), jnp.where(uge, umid, uhi)

    _, hi = lax.fori_loop(0, 8, ubody,
                          (lo, jnp.max(mcls, axis=1, keepdims=True)))

    def bbody(_, carry):
        lo, hi = carry
        mid = 0.5 * (lo + hi)
        cnt = jnp.sum(jnp.where(dist <= mid, 1.0, 0.0), axis=1, keepdims=True)
        ge = cnt >= kf
        return jnp.where(ge, lo, mid), jnp.where(ge, mid, hi)

    _, thr = lax.fori_loop(0, 11, bbody, (lo, hi))

    q_raw = q_ref[0]
    qn = _ln(q_raw, g1_ref[0][None, :], b1_ref[0][None, :])
    wq = w_ref[:d]
    bq = b_ref[0, :d][None, :]
    qp = (lax.dot_general(qn, wq, (((1,), (1,)), ((), ())),
                          preferred_element_type=jnp.float32) + bq) * scale

    kp = kp_ref[0]
    vp = vp_ref[0]
    sbias = jnp.where(dist <= thr, 0.0, NEG)
    ctx_heads = []
    for h in range(H):
        qh = qp[:, h * dh:(h + 1) * dh].astype(jnp.bfloat16)
        kh = kp[:, h * dh:(h + 1) * dh]
        vh = vp[:, h * dh:(h + 1) * dh]
        s = lax.dot_general(qh, kh, (((1,), (1,)), ((), ())),
                            preferred_element_type=jnp.float32)
        e = jnp.exp(s + sbias)
        den = jnp.sum(e, axis=1, keepdims=True)
        ctx = lax.dot_general(e.astype(jnp.bfloat16), vh,
                              (((1,), (0,)), ((), ())),
                              preferred_element_type=jnp.float32) / den
        ctx_heads.append(ctx)
    ctx = jnp.concatenate(ctx_heads, axis=1)

    attended = lax.dot_general(ctx, ow_ref[...], (((1,), (1,)), ((), ())),
                               preferred_element_type=jnp.float32) + ob_ref[0][None, :]
    out1 = q_raw + attended

    hn = _ln(out1, g2_ref[0][None, :], b2_ref[0][None, :])
    h1 = lax.dot_general(hn.astype(jnp.bfloat16), fw1_ref[...].astype(jnp.bfloat16),
                         (((1,), (1,)), ((), ())),
                         preferred_element_type=jnp.float32) + fb1_ref[0][None, :]
    h1 = 0.5 * h1 * (1.0 + lax.erf(h1 * jnp.float32(0.7071067811865476)))
    h2 = lax.dot_general(h1.astype(jnp.bfloat16), fw2_ref[...].astype(jnp.bfloat16),
                         (((1,), (1,)), ((), ())),
                         preferred_element_type=jnp.float32) + fb2_ref[0][None, :]
    out_ref[0] = out1 + h2


def kernel(proxy_feats, proxy_coords, point_feats, point_coords,
           in_proj_w, in_proj_b, out_proj_w, out_proj_b,
           ln1_g, ln1_b, ln2_g, ln2_b, ff_w1, ff_b1, ff_w2, ff_b2):
    B, K, d = proxy_feats.shape
    N = point_coords.shape[1]
    ksel = min(KNN, N)
    NB = min(2048, N)
    BQ = min(256, K)

    qcT = jnp.transpose(proxy_coords, (0, 2, 1))
    qcT = jnp.pad(qcT, ((0, 0), (0, 5), (0, 0)))
    pcT = jnp.transpose(point_coords, (0, 2, 1))
    pcT = jnp.pad(pcT, ((0, 0), (0, 5), (0, 0)))

    b2 = in_proj_b.reshape(1, 3 * d)
    ob2 = out_proj_b.reshape(1, d)
    g12, b12 = ln1_g.reshape(1, d), ln1_b.reshape(1, d)
    g22, b22 = ln2_g.reshape(1, d), ln2_b.reshape(1, d)
    fb12 = ff_b1.reshape(1, 4 * d)
    fb22 = ff_b2.reshape(1, d)

    full = lambda *s: pl.BlockSpec(s, lambda i, j: (0,) * len(s))

    kp_all, vp_all = pl.pallas_call(
        functools.partial(_kv_proj_kernel, d=d),
        grid=(B, N // NB),
        compiler_params=pltpu.CompilerParams(
            dimension_semantics=("parallel", "parallel")),
        in_specs=[
            pl.BlockSpec((1, NB, d), lambda b, n: (b, n, 0)),
            full(3 * d, d),
            full(1, 3 * d),
            full(1, d),
            full(1, d),
        ],
        out_specs=[
            pl.BlockSpec((1, NB, d), lambda b, n: (b, n, 0)),
            pl.BlockSpec((1, NB, d), lambda b, n: (b, n, 0)),
        ],
        out_shape=[
            jax.ShapeDtypeStruct((B, N, d), jnp.bfloat16),
            jax.ShapeDtypeStruct((B, N, d), jnp.bfloat16),
        ],
    )(point_feats, in_proj_w, b2, g12, b12)

    out = pl.pallas_call(
        functools.partial(_attn_kernel, d=d, ksel=ksel),
        grid=(B, K // BQ),
        compiler_params=pltpu.CompilerParams(
            dimension_semantics=("parallel", "parallel")),
        in_specs=[
            pl.BlockSpec((1, BQ, d), lambda b, q: (b, q, 0)),
            pl.BlockSpec((1, 8, BQ), lambda b, q: (b, 0, q)),
            pl.BlockSpec((1, 8, N), lambda b, q: (b, 0, 0)),
            pl.BlockSpec((1, N, d), lambda b, q: (b, 0, 0)),
            pl.BlockSpec((1, N, d), lambda b, q: (b, 0, 0)),
            full(3 * d, d),
            full(1, 3 * d),
            full(d, d),
            full(1, d),
            full(1, d),
            full(1, d),
            full(1, d),
            full(1, d),
            full(4 * d, d),
            full(1, 4 * d),
            full(d, 4 * d),
            full(1, d),
        ],
        out_specs=pl.BlockSpec((1, BQ, d), lambda b, q: (b, q, 0)),
        out_shape=jax.ShapeDtypeStruct((B, K, d), jnp.float32),
    )(proxy_feats, qcT, pcT, kp_all, vp_all,
      in_proj_w, b2, out_proj_w, ob2,
      g12, b12, g22, b22, ff_w1, fb12, ff_w2, fb22)

    return out


# speedup vs baseline: 1.0473x; 1.0473x over previous
import functools
import jax
import jax.numpy as jnp
from jax import lax
from jax.experimental import pallas as pl
from jax.experimental.pallas import tpu as pltpu

H = 4
KNN = 32
EPS = 1e-5
BIG = 3.0e38
NEG = -3.0e38


def _ln(x, g, b):
    m = jnp.mean(x, axis=-1, keepdims=True)
    v = jnp.mean((x - m) * (x - m), axis=-1, keepdims=True)
    return (x - m) / jnp.sqrt(v + EPS) * g + b


def _kv_proj_kernel(pf_ref, w_ref, b_ref, g_ref, bb_ref, kp_ref, vp_ref, *, d):
    x = pf_ref[0]
    xn = _ln(x, g_ref[0][None, :], bb_ref[0][None, :])
    wk = w_ref[d:2 * d]
    wv = w_ref[2 * d:3 * d]
    bk = b_ref[0, d:2 * d][None, :]
    bv = b_ref[0, 2 * d:3 * d][None, :]
    kp_ref[0] = (lax.dot_general(xn, wk, (((1,), (1,)), ((), ())),
                                 preferred_element_type=jnp.float32)
                 + bk).astype(jnp.bfloat16)
    vp_ref[0] = (lax.dot_general(xn, wv, (((1,), (1,)), ((), ())),
                                 preferred_element_type=jnp.float32)
                 + bv).astype(jnp.bfloat16)


def _attn_kernel(q_ref, qc_ref, pc_ref, kp_ref, vp_ref,
                 w_ref, b_ref, ow_ref, ob_ref,
                 g1_ref, b1_ref, g2_ref, b2_ref,
                 fw1_ref, fb1_ref, fw2_ref, fb2_ref,
                 out_ref, *, d, ksel):
    dh = d // H
    scale = 1.0 / jnp.sqrt(jnp.float32(dh))

    qc = qc_ref[0]
    pc = pc_ref[0]
    pn = jnp.sum(pc * pc, axis=0, keepdims=True)
    cross = lax.dot_general(qc, pc, (((0,), (0,)), ((), ())),
                            preferred_element_type=jnp.float32)
    dist = pn - 2.0 * cross

    ncls = dist.shape[1]
    mcls = dist
    while ncls > 128:
        ncls //= 2
        mcls = jnp.minimum(mcls[:, :ncls], mcls[:, ncls:])
    lo = jnp.min(mcls, axis=1, keepdims=True)
    kf = jnp.float32(ksel)

    def ubody(_, carry):
        ulo, uhi = carry
        umid = 0.5 * (ulo + uhi)
        ucnt = jnp.sum(jnp.where(mcls <= umid, 1.0, 0.0), axis=1,
                       keepdims=True)
        uge = ucnt >= kf
        return jnp.where(uge, ulo, umid), jnp.where(uge, umid, uhi)

    _, hi = lax.fori_loop(0, 8, ubody,
                          (lo, jnp.max(mcls, axis=1, keepdims=True)))

    def bbody(_, carry):
        lo, hi = carry
        mid = 0.5 * (lo + hi)
        cnt = jnp.sum(jnp.where(dist <= mid, 1.0, 0.0), axis=1, keepdims=True)
        ge = cnt >= kf
        return jnp.where(ge, lo, mid), jnp.where(ge, mid, hi)

    _, thr = lax.fori_loop(0, 11, bbody, (lo, hi))

    q_raw = q_ref[0]
    qn = _ln(q_raw, g1_ref[0][None, :], b1_ref[0][None, :])
    wq = w_ref[:d]
    bq = b_ref[0, :d][None, :]
    qp = (lax.dot_general(qn, wq, (((1,), (1,)), ((), ())),
                          preferred_element_type=jnp.float32) + bq) * scale

    kp = kp_ref[0]
    vp = vp_ref[0]
    sbias = jnp.where(dist <= thr, 0.0, NEG)
    ctx_heads = []
    for h in range(H):
        qh = qp[:, h * dh:(h + 1) * dh].astype(jnp.bfloat16)
        kh = kp[:, h * dh:(h + 1) * dh]
        vh = vp[:, h * dh:(h + 1) * dh]
        s = lax.dot_general(qh, kh, (((1,), (1,)), ((), ())),
                            preferred_element_type=jnp.float32)
        e = jnp.exp(s + sbias)
        den = jnp.sum(e, axis=1, keepdims=True)
        ctx = lax.dot_general(e.astype(jnp.bfloat16), vh,
                              (((1,), (0,)), ((), ())),
                              preferred_element_type=jnp.float32) / den
        ctx_heads.append(ctx)
    ctx = jnp.concatenate(ctx_heads, axis=1)

    attended = lax.dot_general(ctx, ow_ref[...], (((1,), (1,)), ((), ())),
                               preferred_element_type=jnp.float32) + ob_ref[0][None, :]
    out1 = q_raw + attended

    hn = _ln(out1, g2_ref[0][None, :], b2_ref[0][None, :])
    h1 = lax.dot_general(hn.astype(jnp.bfloat16), fw1_ref[...].astype(jnp.bfloat16),
                         (((1,), (1,)), ((), ())),
                         preferred_element_type=jnp.float32) + fb1_ref[0][None, :]
    h1 = 0.5 * h1 * (1.0 + lax.erf(h1 * jnp.float32(0.7071067811865476)))
    h2 = lax.dot_general(h1.astype(jnp.bfloat16), fw2_ref[...].astype(jnp.bfloat16),
                         (((1,), (1,)), ((), ())),
                         preferred_element_type=jnp.float32) + fb2_ref[0][None, :]
    out_ref[0] = out1 + h2


def kernel(proxy_feats, proxy_coords, point_feats, point_coords,
           in_proj_w, in_proj_b, out_proj_w, out_proj_b,
           ln1_g, ln1_b, ln2_g, ln2_b, ff_w1, ff_b1, ff_w2, ff_b2):
    B, K, d = proxy_feats.shape
    N = point_coords.shape[1]
    ksel = min(KNN, N)
    NB = min(2048, N)
    BQ = min(512, K)

    qcT = jnp.transpose(proxy_coords, (0, 2, 1))
    qcT = jnp.pad(qcT, ((0, 0), (0, 5), (0, 0)))
    pcT = jnp.transpose(point_coords, (0, 2, 1))
    pcT = jnp.pad(pcT, ((0, 0), (0, 5), (0, 0)))

    b2 = in_proj_b.reshape(1, 3 * d)
    ob2 = out_proj_b.reshape(1, d)
    g12, b12 = ln1_g.reshape(1, d), ln1_b.reshape(1, d)
    g22, b22 = ln2_g.reshape(1, d), ln2_b.reshape(1, d)
    fb12 = ff_b1.reshape(1, 4 * d)
    fb22 = ff_b2.reshape(1, d)

    full = lambda *s: pl.BlockSpec(s, lambda i, j: (0,) * len(s))

    kp_all, vp_all = pl.pallas_call(
        functools.partial(_kv_proj_kernel, d=d),
        grid=(B, N // NB),
        compiler_params=pltpu.CompilerParams(
            dimension_semantics=("parallel", "parallel")),
        in_specs=[
            pl.BlockSpec((1, NB, d), lambda b, n: (b, n, 0)),
            full(3 * d, d),
            full(1, 3 * d),
            full(1, d),
            full(1, d),
        ],
        out_specs=[
            pl.BlockSpec((1, NB, d), lambda b, n: (b, n, 0)),
            pl.BlockSpec((1, NB, d), lambda b, n: (b, n, 0)),
        ],
        out_shape=[
            jax.ShapeDtypeStruct((B, N, d), jnp.bfloat16),
            jax.ShapeDtypeStruct((B, N, d), jnp.bfloat16),
        ],
    )(point_feats, in_proj_w, b2, g12, b12)

    out = pl.pallas_call(
        functools.partial(_attn_kernel, d=d, ksel=ksel),
        grid=(B, K // BQ),
        compiler_params=pltpu.CompilerParams(
            dimension_semantics=("parallel", "parallel"),
            vmem_limit_bytes=100 * 1024 * 1024),
        in_specs=[
            pl.BlockSpec((1, BQ, d), lambda b, q: (b, q, 0)),
            pl.BlockSpec((1, 8, BQ), lambda b, q: (b, 0, q)),
            pl.BlockSpec((1, 8, N), lambda b, q: (b, 0, 0)),
            pl.BlockSpec((1, N, d), lambda b, q: (b, 0, 0)),
            pl.BlockSpec((1, N, d), lambda b, q: (b, 0, 0)),
            full(3 * d, d),
            full(1, 3 * d),
            full(d, d),
            full(1, d),
            full(1, d),
            full(1, d),
            full(1, d),
            full(1, d),
            full(4 * d, d),
            full(1, 4 * d),
            full(d, 4 * d),
            full(1, d),
        ],
        out_specs=pl.BlockSpec((1, BQ, d), lambda b, q: (b, q, 0)),
        out_shape=jax.ShapeDtypeStruct((B, K, d), jnp.float32),
    )(proxy_feats, qcT, pcT, kp_all, vp_all,
      in_proj_w, b2, out_proj_w, ob2,
      g12, b12, g22, b22, ff_w1, fb12, ff_w2, fb22)

    return out


# speedup vs baseline: 1.0811x; 1.0323x over previous
import functools
import jax
import jax.numpy as jnp
from jax import lax
from jax.experimental import pallas as pl
from jax.experimental.pallas import tpu as pltpu

H = 4
KNN = 32
EPS = 1e-5
BIG = 3.0e38
NEG = -3.0e38


def _ln(x, g, b):
    m = jnp.mean(x, axis=-1, keepdims=True)
    v = jnp.mean((x - m) * (x - m), axis=-1, keepdims=True)
    return (x - m) / jnp.sqrt(v + EPS) * g + b


def _kv_proj_kernel(pf_ref, w_ref, b_ref, g_ref, bb_ref, kp_ref, vp_ref, *, d):
    x = pf_ref[0]
    xn = _ln(x, g_ref[0][None, :], bb_ref[0][None, :])
    wk = w_ref[d:2 * d]
    wv = w_ref[2 * d:3 * d]
    bk = b_ref[0, d:2 * d][None, :]
    bv = b_ref[0, 2 * d:3 * d][None, :]
    kp_ref[0] = (lax.dot_general(xn, wk, (((1,), (1,)), ((), ())),
                                 preferred_element_type=jnp.float32)
                 + bk).astype(jnp.bfloat16)
    vp_ref[0] = (lax.dot_general(xn, wv, (((1,), (1,)), ((), ())),
                                 preferred_element_type=jnp.float32)
                 + bv).astype(jnp.bfloat16)


def _attn_kernel(q_ref, qc_ref, pc_ref, kp_ref, vp_ref,
                 w_ref, b_ref, ow_ref, ob_ref,
                 g1_ref, b1_ref, g2_ref, b2_ref,
                 fw1_ref, fb1_ref, fw2_ref, fb2_ref,
                 out_ref, *, d, ksel):
    dh = d // H
    scale = 1.0 / jnp.sqrt(jnp.float32(dh))

    qc = qc_ref[0]
    pc = pc_ref[0]
    pn = jnp.sum(pc * pc, axis=0, keepdims=True)
    cross = lax.dot_general(qc, pc, (((0,), (0,)), ((), ())),
                            preferred_element_type=jnp.float32)
    dist = pn - 2.0 * cross

    ncls = dist.shape[1]
    mcls = dist
    while ncls > 128:
        ncls //= 2
        mcls = jnp.minimum(mcls[:, :ncls], mcls[:, ncls:])
    lo = jnp.min(mcls, axis=1, keepdims=True)
    kf = jnp.float32(ksel)

    def ubody(_, carry):
        ulo, uhi = carry
        umid = 0.5 * (ulo + uhi)
        ucnt = jnp.sum(jnp.where(mcls <= umid, 1.0, 0.0), axis=1,
                       keepdims=True)
        uge = ucnt >= kf
        return jnp.where(uge, ulo, umid), jnp.where(uge, umid, uhi)

    _, hi = lax.fori_loop(0, 8, ubody,
                          (lo, jnp.max(mcls, axis=1, keepdims=True)))

    def bbody(_, carry):
        lo, hi = carry
        mid = 0.5 * (lo + hi)
        cnt = jnp.sum(jnp.where(dist <= mid, 1.0, 0.0), axis=1, keepdims=True)
        ge = cnt >= kf
        return jnp.where(ge, lo, mid), jnp.where(ge, mid, hi)

    _, thr = lax.fori_loop(0, 10, bbody, (lo, hi))

    q_raw = q_ref[0]
    qn = _ln(q_raw, g1_ref[0][None, :], b1_ref[0][None, :])
    wq = w_ref[:d]
    bq = b_ref[0, :d][None, :]
    qp = (lax.dot_general(qn, wq, (((1,), (1,)), ((), ())),
                          preferred_element_type=jnp.float32) + bq) * scale

    kp = kp_ref[0]
    vp = vp_ref[0]
    sbias = jnp.where(dist <= thr, 0.0, NEG)
    ctx_heads = []
    for h in range(H):
        qh = qp[:, h * dh:(h + 1) * dh].astype(jnp.bfloat16)
        kh = kp[:, h * dh:(h + 1) * dh]
        vh = vp[:, h * dh:(h + 1) * dh]
        s = lax.dot_general(qh, kh, (((1,), (1,)), ((), ())),
                            preferred_element_type=jnp.float32)
        e = jnp.exp(s + sbias)
        den = jnp.sum(e, axis=1, keepdims=True)
        ctx = lax.dot_general(e.astype(jnp.bfloat16), vh,
                              (((1,), (0,)), ((), ())),
                              preferred_element_type=jnp.float32) / den
        ctx_heads.append(ctx)
    ctx = jnp.concatenate(ctx_heads, axis=1)

    attended = lax.dot_general(ctx, ow_ref[...], (((1,), (1,)), ((), ())),
                               preferred_element_type=jnp.float32) + ob_ref[0][None, :]
    out1 = q_raw + attended

    hn = _ln(out1, g2_ref[0][None, :], b2_ref[0][None, :])
    h1 = lax.dot_general(hn.astype(jnp.bfloat16), fw1_ref[...].astype(jnp.bfloat16),
                         (((1,), (1,)), ((), ())),
                         preferred_element_type=jnp.float32) + fb1_ref[0][None, :]
    h1 = 0.5 * h1 * (1.0 + lax.erf(h1 * jnp.float32(0.7071067811865476)))
    h2 = lax.dot_general(h1.astype(jnp.bfloat16), fw2_ref[...].astype(jnp.bfloat16),
                         (((1,), (1,)), ((), ())),
                         preferred_element_type=jnp.float32) + fb2_ref[0][None, :]
    out_ref[0] = out1 + h2


def kernel(proxy_feats, proxy_coords, point_feats, point_coords,
           in_proj_w, in_proj_b, out_proj_w, out_proj_b,
           ln1_g, ln1_b, ln2_g, ln2_b, ff_w1, ff_b1, ff_w2, ff_b2):
    B, K, d = proxy_feats.shape
    N = point_coords.shape[1]
    ksel = min(KNN, N)
    NB = min(2048, N)
    BQ = min(512, K)

    qcT = jnp.transpose(proxy_coords, (0, 2, 1))
    qcT = jnp.pad(qcT, ((0, 0), (0, 5), (0, 0)))
    pcT = jnp.transpose(point_coords, (0, 2, 1))
    pcT = jnp.pad(pcT, ((0, 0), (0, 5), (0, 0)))

    b2 = in_proj_b.reshape(1, 3 * d)
    ob2 = out_proj_b.reshape(1, d)
    g12, b12 = ln1_g.reshape(1, d), ln1_b.reshape(1, d)
    g22, b22 = ln2_g.reshape(1, d), ln2_b.reshape(1, d)
    fb12 = ff_b1.reshape(1, 4 * d)
    fb22 = ff_b2.reshape(1, d)

    full = lambda *s: pl.BlockSpec(s, lambda i, j: (0,) * len(s))

    kp_all, vp_all = pl.pallas_call(
        functools.partial(_kv_proj_kernel, d=d),
        grid=(B, N // NB),
        compiler_params=pltpu.CompilerParams(
            dimension_semantics=("parallel", "parallel")),
        in_specs=[
            pl.BlockSpec((1, NB, d), lambda b, n: (b, n, 0)),
            full(3 * d, d),
            full(1, 3 * d),
            full(1, d),
            full(1, d),
        ],
        out_specs=[
            pl.BlockSpec((1, NB, d), lambda b, n: (b, n, 0)),
            pl.BlockSpec((1, NB, d), lambda b, n: (b, n, 0)),
        ],
        out_shape=[
            jax.ShapeDtypeStruct((B, N, d), jnp.bfloat16),
            jax.ShapeDtypeStruct((B, N, d), jnp.bfloat16),
        ],
    )(point_feats, in_proj_w, b2, g12, b12)

    out = pl.pallas_call(
        functools.partial(_attn_kernel, d=d, ksel=ksel),
        grid=(B, K // BQ),
        compiler_params=pltpu.CompilerParams(
            dimension_semantics=("parallel", "parallel"),
            vmem_limit_bytes=100 * 1024 * 1024),
        in_specs=[
            pl.BlockSpec((1, BQ, d), lambda b, q: (b, q, 0)),
            pl.BlockSpec((1, 8, BQ), lambda b, q: (b, 0, q)),
            pl.BlockSpec((1, 8, N), lambda b, q: (b, 0, 0)),
            pl.BlockSpec((1, N, d), lambda b, q: (b, 0, 0)),
            pl.BlockSpec((1, N, d), lambda b, q: (b, 0, 0)),
            full(3 * d, d),
            full(1, 3 * d),
            full(d, d),
            full(1, d),
            full(1, d),
            full(1, d),
            full(1, d),
            full(1, d),
            full(4 * d, d),
            full(1, 4 * d),
            full(d, 4 * d),
            full(1, d),
        ],
        out_specs=pl.BlockSpec((1, BQ, d), lambda b, q: (b, q, 0)),
        out_shape=jax.ShapeDtypeStruct((B, K, d), jnp.float32),
    )(proxy_feats, qcT, pcT, kp_all, vp_all,
      in_proj_w, b2, out_proj_w, ob2,
      g12, b12, g22, b22, ff_w1, fb12, ff_w2, fb22)

    return out
